# baseline (speedup 1.0000x reference)
import functools

import jax
import jax.numpy as jnp
from jax import lax
from jax.experimental import pallas as pl
from jax.experimental.pallas import tpu as pltpu
from jax.experimental.pallas import tpu_sc as plsc

_W = 256
_L = 16
_NC, _NS = 2, 16
_NW = _NC * _NS


@functools.lru_cache(maxsize=None)
def _build_sc_call(B, P):
    assert B % _NW == 0
    assert P % 128 == 0
    spw = B // _NW
    chunks = P // _L
    rows = 2 * P // 128

    mesh = plsc.VectorSubcoreMesh(core_axis_name="c", subcore_axis_name="s")

    @functools.partial(
        pl.kernel,
        out_type=jax.ShapeDtypeStruct((_NW, _L), jnp.float32),
        mesh=mesh,
        compiler_params=pltpu.CompilerParams(needs_layout_passes=False),
        scratch_types=[
            pltpu.VMEM((2 * _W, 128), jnp.float32),
            pltpu.VMEM((rows, 128), jnp.float32),
            pltpu.VMEM((_L,), jnp.float32),
            pltpu.SemaphoreType.DMA,
            pltpu.SemaphoreType.DMA,
        ],
    )
    def sc_call(y_hbm, d_hbm, out_hbm, dv, yv, acc_v, sem_d, sem_y):
        wid = lax.axis_index("s") * _NC + lax.axis_index("c")
        base = wid * spw

        pltpu.async_copy(d_hbm.at[base], dv, sem_d)
        pltpu.async_copy(y_hbm.at[base], yv, sem_y)

        def sample_body(s, carry):
            pltpu.make_async_copy(d_hbm.at[base + s], dv, sem_d).wait()
            pltpu.make_async_copy(y_hbm.at[base + s], yv, sem_y).wait()

            @plsc.parallel_loop(0, chunks, unroll=8, carry=carry)
            def loop_acc(i, c):
                a_sq, a_g = c
                row = 2 * (i // 8)
                col = _L * (i % 8)
                x = yv[row, pl.ds(col, _L)]
                y = yv[row + 1, pl.ds(col, _L)]
                ci = (x * 256.0).astype(jnp.int32)
                ri = (y * 256.0).astype(jnp.int32)
                q = ri + (ri & ~7) + ((ci & 128) >> 4)
                l = ci & 127
                g = plsc.load_gather(dv, [q, l])
                return a_sq + g * g, a_g + g

            @pl.when(s < spw - 1)
            def _():
                pltpu.async_copy(d_hbm.at[base + s + 1], dv, sem_d)
                pltpu.async_copy(y_hbm.at[base + s + 1], yv, sem_y)

            return loop_acc

        zero = jnp.zeros((_L,), jnp.float32)
        ssq, sg = lax.fori_loop(0, spw, sample_body, (zero, zero))

        npts = float(spw * chunks)
        acc_v[...] = 262144.0 * ssq - 260096.0 * sg + npts * 64516.0
        pltpu.sync_copy(acc_v, out_hbm.at[wid])

    return sc_call


def kernel(y_pred, distance_maps):
    B, P = y_pred.shape[0], y_pred.shape[1]
    y3 = (y_pred.reshape(B, P // 128, 128, 2)
          .transpose(0, 1, 3, 2)
          .reshape(B, 2 * P // 128, 128))
    d3 = (distance_maps.reshape(B, _W // 8, 8, 2, 128)
          .transpose(0, 1, 3, 2, 4)
          .reshape(B, 2 * _W, 128))
    partial = _build_sc_call(B, P)(y3, d3)
    return jnp.sum(partial) * (1.0 / (B * P))


# speedup vs baseline: 1.0643x; 1.0643x over previous
import functools

import jax
import jax.numpy as jnp
from jax import lax
from jax.experimental import pallas as pl
from jax.experimental.pallas import tpu as pltpu
from jax.experimental.pallas import tpu_sc as plsc

_W = 256
_L = 16
_NC, _NS = 2, 16
_NW = _NC * _NS


@functools.lru_cache(maxsize=None)
def _build_sc_call(B, P):
    assert B % _NW == 0
    assert P % 128 == 0
    spw = B // _NW
    chunks = P // _L
    rows = 2 * P // 128

    mesh = plsc.VectorSubcoreMesh(core_axis_name="c", subcore_axis_name="s")

    @functools.partial(
        pl.kernel,
        out_type=jax.ShapeDtypeStruct((_NW, _L), jnp.float32),
        mesh=mesh,
        compiler_params=pltpu.CompilerParams(needs_layout_passes=False),
        scratch_types=[
            pltpu.VMEM((2 * _W, 128), jnp.float32),
            pltpu.VMEM((rows, 128), jnp.float32),
            pltpu.VMEM((rows, 128), jnp.float32),
            pltpu.VMEM((_L,), jnp.float32),
            pltpu.SemaphoreType.DMA,
            pltpu.SemaphoreType.DMA,
        ],
    )
    def sc_call(y_hbm, d_hbm, out_hbm, dv, ya, yb, acc_v, sem_d, sem_y):
        wid = lax.axis_index("s") * _NC + lax.axis_index("c")
        base = wid * spw

        def sample_loss(yv, carry):
            @plsc.parallel_loop(0, chunks, unroll=8, carry=carry)
            def loop_acc(i, c):
                a_sq, a_g = c
                row = 2 * (i // 8)
                col = _L * (i % 8)
                x = yv[row, pl.ds(col, _L)]
                y = yv[row + 1, pl.ds(col, _L)]
                ci = (x * 256.0).astype(jnp.int32)
                ri = (y * 256.0).astype(jnp.int32)
                q = ri + (ri & ~7) + ((ci & 128) >> 4)
                l = ci & 127
                g = plsc.load_gather(dv, [q, l])
                return a_sq + g * g, a_g + g

            return loop_acc

        pltpu.async_copy(d_hbm.at[base], dv, sem_d)
        pltpu.async_copy(y_hbm.at[base], ya, sem_y)

        def pair_body(j, carry):
            e = base + 2 * j
            pltpu.make_async_copy(d_hbm.at[e], dv, sem_d).wait()
            pltpu.make_async_copy(y_hbm.at[e], ya, sem_y).wait()
            pltpu.async_copy(y_hbm.at[e + 1], yb, sem_y)
            carry = sample_loss(ya, carry)
            pltpu.async_copy(d_hbm.at[e + 1], dv, sem_d)
            pltpu.make_async_copy(d_hbm.at[e + 1], dv, sem_d).wait()
            pltpu.make_async_copy(y_hbm.at[e + 1], yb, sem_y).wait()

            @pl.when(j < spw // 2 - 1)
            def _():
                pltpu.async_copy(y_hbm.at[e + 2], ya, sem_y)

            carry = sample_loss(yb, carry)

            @pl.when(j < spw // 2 - 1)
            def _():
                pltpu.async_copy(d_hbm.at[e + 2], dv, sem_d)

            return carry

        zero = jnp.zeros((_L,), jnp.float32)
        ssq, sg = lax.fori_loop(0, spw // 2, pair_body, (zero, zero))

        npts = float(spw * chunks)
        acc_v[...] = 262144.0 * ssq - 260096.0 * sg + npts * 64516.0
        pltpu.sync_copy(acc_v, out_hbm.at[wid])

    return sc_call


def kernel(y_pred, distance_maps):
    B, P = y_pred.shape[0], y_pred.shape[1]
    y3 = (y_pred.reshape(B, P // 128, 128, 2)
          .transpose(0, 1, 3, 2)
          .reshape(B, 2 * P // 128, 128))
    d3 = (distance_maps.reshape(B, _W // 8, 8, 2, 128)
          .transpose(0, 1, 3, 2, 4)
          .reshape(B, 2 * _W, 128))
    partial = _build_sc_call(B, P)(y3, d3)
    return jnp.sum(partial) * (1.0 / (B * P))
